# baseline (speedup 1.0000x reference)
import functools

import jax
import jax.numpy as jnp
from jax import lax
from jax.experimental import pallas as pl
from jax.experimental.pallas import tpu as pltpu
from jax.experimental.pallas import tpu_sc as plsc

RANK = 32
LANES = 16
CHUNK = 128


@functools.partial(jax.jit, static_argnums=(4, 5))
def _mf_dot(user2d, anime2d, Ut, Vt, batch, n_workers):
    rows_per_w = batch // n_workers
    n_chunks = rows_per_w // CHUNK
    n_groups = rows_per_w // LANES
    mesh = plsc.VectorSubcoreMesh(core_axis_name="c", subcore_axis_name="s")
    info = plsc.get_sparse_core_info()
    nc = info.num_cores

    @functools.partial(
        pl.kernel,
        mesh=mesh,
        compiler_params=pltpu.CompilerParams(
            needs_layout_passes=False, use_tc_tiling_on_sc=False),
        out_type=jax.ShapeDtypeStruct((batch,), jnp.float32),
        scratch_types=[
            pltpu.VMEM((n_chunks, CHUNK), jnp.int32),
            pltpu.VMEM((n_chunks, CHUNK), jnp.int32),
            pltpu.VMEM((RANK, rows_per_w), jnp.float32),
            pltpu.VMEM((RANK, rows_per_w), jnp.float32),
            pltpu.VMEM((rows_per_w,), jnp.float32),
            pltpu.SemaphoreType.DMA,
        ],
    )
    def body(user_hbm, anime_hbm, ut_hbm, vt_hbm, out_hbm,
             uidx, vidx, ucols, vcols, outv, sem):
        wid = lax.axis_index("s") * nc + lax.axis_index("c")
        base = pl.multiple_of(wid * rows_per_w, rows_per_w)

        pltpu.sync_copy(user_hbm.at[wid], uidx)
        pltpu.sync_copy(anime_hbm.at[wid], vidx)

        copies = []
        for k in range(RANK):
            for j in range(n_chunks):
                copies.append(pltpu.async_copy(
                    ut_hbm.at[k].at[uidx.at[j]],
                    ucols.at[k, pl.ds(j * CHUNK, CHUNK)], sem))
                copies.append(pltpu.async_copy(
                    vt_hbm.at[k].at[vidx.at[j]],
                    vcols.at[k, pl.ds(j * CHUNK, CHUNK)], sem))
        for c in copies:
            c.wait()

        def group(g, carry):
            acc = jnp.zeros((LANES,), jnp.float32)
            for k in range(RANK):
                uu = ucols[k, pl.ds(g * LANES, LANES)]
                vv = vcols[k, pl.ds(g * LANES, LANES)]
                acc = acc + uu * vv
            outv[pl.ds(g * LANES, LANES)] = acc
            return carry

        lax.fori_loop(0, n_groups, group, 0)
        pltpu.sync_copy(outv, out_hbm.at[pl.ds(base, rows_per_w)])

    return body(user2d, anime2d, Ut, Vt)


def kernel(user, anime, U, V):
    batch = user.shape[0]
    n_workers = 32
    n_chunks = (batch // n_workers) // CHUNK
    user2d = user.astype(jnp.int32).reshape(n_workers, n_chunks, CHUNK)
    anime2d = anime.astype(jnp.int32).reshape(n_workers, n_chunks, CHUNK)
    return _mf_dot(user2d, anime2d, U.T, V.T, batch, n_workers)


# speedup vs baseline: 10.7493x; 10.7493x over previous
import functools

import jax
import jax.numpy as jnp
from jax import lax
from jax.experimental import pallas as pl
from jax.experimental.pallas import tpu as pltpu
from jax.experimental.pallas import tpu_sc as plsc

RANK = 32
LANES = 16
GSIZE = 4
WIN = 128


def _take16(vec, idx):
    return vec.at[idx].get(mode="promise_in_bounds")


@functools.partial(jax.jit, static_argnums=(4, 5))
def _mf_dot(user2d, anime2d, Ut, Vt, batch, n_workers):
    rows_per_w = batch // n_workers
    n_groups = rows_per_w // GSIZE
    mesh = plsc.VectorSubcoreMesh(core_axis_name="c", subcore_axis_name="s")
    info = plsc.get_sparse_core_info()
    nc = info.num_cores

    @functools.partial(
        pl.kernel,
        mesh=mesh,
        compiler_params=pltpu.CompilerParams(
            needs_layout_passes=False, use_tc_tiling_on_sc=True),
        out_type=jax.ShapeDtypeStruct((batch,), jnp.float32),
        scratch_types=[
            pltpu.VMEM((rows_per_w + LANES,), jnp.int32),
            pltpu.VMEM((rows_per_w + LANES,), jnp.int32),
            pltpu.VMEM((GSIZE, RANK, WIN), jnp.float32),
            pltpu.VMEM((GSIZE, RANK, WIN), jnp.float32),
            pltpu.VMEM((GSIZE, RANK, WIN), jnp.float32),
            pltpu.VMEM((GSIZE, RANK, WIN), jnp.float32),
            pltpu.VMEM((rows_per_w + LANES,), jnp.float32),
            pltpu.SemaphoreType.DMA,
            pltpu.SemaphoreType.DMA,
        ],
    )
    def body(user_hbm, anime_hbm, ut_hbm, vt_hbm, out_hbm,
             uidx, vidx, ubufa, ubufb, vbufa, vbufb, outv, sema, semb):
        wid = lax.axis_index("s") * nc + lax.axis_index("c")
        base = pl.multiple_of(wid * rows_per_w, rows_per_w)

        pltpu.sync_copy(user_hbm.at[wid], uidx.at[pl.ds(0, rows_per_w)])
        pltpu.sync_copy(anime_hbm.at[wid], vidx.at[pl.ds(0, rows_per_w)])

        lane = lax.iota(jnp.int32, LANES)
        item = lane // GSIZE
        kk = lane % GSIZE

        def enqueue_group(g, ubuf, vbuf, sem):
            uvec = uidx[pl.ds(g * GSIZE, LANES)]
            vvec = vidx[pl.ds(g * GSIZE, LANES)]
            for l in range(GSIZE):
                ucol = pl.multiple_of((uvec[l] // WIN) * WIN, WIN)
                vcol = pl.multiple_of((vvec[l] // WIN) * WIN, WIN)
                pltpu.async_copy(
                    ut_hbm.at[:, pl.ds(ucol, WIN)], ubuf.at[l], sem)
                pltpu.async_copy(
                    vt_hbm.at[:, pl.ds(vcol, WIN)], vbuf.at[l], sem)

        def drain_group(ubuf, vbuf, sem):
            for l in range(GSIZE):
                pltpu.make_async_copy(
                    ut_hbm.at[:, pl.ds(0, WIN)], ubuf.at[l], sem).wait()
                pltpu.make_async_copy(
                    ut_hbm.at[:, pl.ds(0, WIN)], vbuf.at[l], sem).wait()

        def compute_group(g, ubuf, vbuf):
            uvec = uidx[pl.ds(g * GSIZE, LANES)]
            vvec = vidx[pl.ds(g * GSIZE, LANES)]
            ux = _take16(uvec, item) % WIN
            vx = _take16(vvec, item) % WIN
            acc = jnp.zeros((LANES,), jnp.float32)
            for j in range(RANK // GSIZE):
                krow = j * GSIZE + kk
                gu = plsc.load_gather(ubuf, [item, krow, ux])
                gv = plsc.load_gather(vbuf, [item, krow, vx])
                acc = acc + gu * gv
            acc = acc + _take16(acc, lane ^ 1)
            acc = acc + _take16(acc, lane ^ 2)
            res = _take16(acc, lane * GSIZE)
            plsc.store_compressed(
                outv.at[pl.ds(g * GSIZE, LANES)], res, mask=lane < GSIZE)

        enqueue_group(0, ubufa, vbufa, sema)

        def step(t, carry):
            g0 = 2 * t
            enqueue_group(g0 + 1, ubufb, vbufb, semb)
            drain_group(ubufa, vbufa, sema)
            compute_group(g0, ubufa, vbufa)

            @pl.when(t < n_groups // 2 - 1)
            def _():
                enqueue_group(g0 + 2, ubufa, vbufa, sema)
            drain_group(ubufb, vbufb, semb)
            compute_group(g0 + 1, ubufb, vbufb)
            return carry

        lax.fori_loop(0, n_groups // 2, step, 0)
        pltpu.sync_copy(outv.at[pl.ds(0, rows_per_w)],
                        out_hbm.at[pl.ds(base, rows_per_w)])

    return body(user2d, anime2d, Ut, Vt)


def kernel(user, anime, U, V):
    batch = user.shape[0]
    n_workers = 32
    user2d = user.astype(jnp.int32).reshape(n_workers, batch // n_workers)
    anime2d = anime.astype(jnp.int32).reshape(n_workers, batch // n_workers)
    return _mf_dot(user2d, anime2d, U.T, V.T, batch, n_workers)


# speedup vs baseline: 12.5605x; 1.1685x over previous
import functools

import jax
import jax.numpy as jnp
from jax import lax
from jax.experimental import pallas as pl
from jax.experimental.pallas import tpu as pltpu
from jax.experimental.pallas import tpu_sc as plsc

RANK = 32
LANES = 16
GSIZE = 4
WIN = 128
CHUNK = 128


def _take16(vec, idx):
    return vec.at[idx].get(mode="promise_in_bounds")


@functools.partial(jax.jit, static_argnums=(2,))
def _v_gather(anime2d, V, n_workers):
    batch = anime2d.shape[0] * anime2d.shape[1] * anime2d.shape[2]
    rows_per_w = batch // n_workers
    n_chunks = rows_per_w // CHUNK
    mesh = plsc.VectorSubcoreMesh(core_axis_name="c", subcore_axis_name="s")
    info = plsc.get_sparse_core_info()
    nc = info.num_cores

    @functools.partial(
        pl.kernel,
        mesh=mesh,
        compiler_params=pltpu.CompilerParams(
            needs_layout_passes=False, use_tc_tiling_on_sc=False),
        out_type=jax.ShapeDtypeStruct((batch, RANK), jnp.float32),
        scratch_types=[
            pltpu.VMEM((n_chunks, CHUNK), jnp.int32),
            pltpu.VMEM((rows_per_w, RANK), jnp.float32),
            pltpu.SemaphoreType.DMA,
        ],
    )
    def body(anime_hbm, v_hbm, out_hbm, vidx, vrows, sem):
        wid = lax.axis_index("s") * nc + lax.axis_index("c")
        base = pl.multiple_of(wid * rows_per_w, rows_per_w)
        pltpu.sync_copy(anime_hbm.at[wid], vidx)
        copies = []
        for j in range(n_chunks):
            copies.append(pltpu.async_copy(
                v_hbm.at[vidx.at[j]], vrows.at[pl.ds(j * CHUNK, CHUNK)], sem))
        for c in copies:
            c.wait()
        pltpu.sync_copy(vrows, out_hbm.at[pl.ds(base, rows_per_w)])

    return body(anime2d, V)


@functools.partial(jax.jit, static_argnums=(3, 4))
def _u_dot(user2d, Ut, vsel, batch, n_workers):
    rows_per_w = batch // n_workers
    n_groups = rows_per_w // GSIZE
    mesh = plsc.VectorSubcoreMesh(core_axis_name="c", subcore_axis_name="s")
    info = plsc.get_sparse_core_info()
    nc = info.num_cores

    @functools.partial(
        pl.kernel,
        mesh=mesh,
        compiler_params=pltpu.CompilerParams(
            needs_layout_passes=False, use_tc_tiling_on_sc=True),
        out_type=jax.ShapeDtypeStruct((batch,), jnp.float32),
        scratch_types=[
            pltpu.VMEM((rows_per_w + LANES,), jnp.int32),
            pltpu.VMEM((GSIZE, RANK, WIN), jnp.float32),
            pltpu.VMEM((GSIZE, RANK, WIN), jnp.float32),
            pltpu.VMEM((rows_per_w, RANK), jnp.float32),
            pltpu.VMEM((rows_per_w + LANES,), jnp.float32),
            pltpu.SemaphoreType.DMA,
            pltpu.SemaphoreType.DMA,
        ],
    )
    def body(user_hbm, ut_hbm, vsel_hbm, out_hbm,
             uidx, ubufa, ubufb, vrows, outv, sema, semb):
        wid = lax.axis_index("s") * nc + lax.axis_index("c")
        base = pl.multiple_of(wid * rows_per_w, rows_per_w)

        pltpu.sync_copy(user_hbm.at[wid], uidx.at[pl.ds(0, rows_per_w)])
        pltpu.sync_copy(vsel_hbm.at[pl.ds(base, rows_per_w)], vrows)

        lane = lax.iota(jnp.int32, LANES)
        item = lane // GSIZE
        kk = lane % GSIZE

        def enqueue_group(g, ubuf, sem):
            uvec = uidx[pl.ds(g * GSIZE, LANES)]
            for l in range(GSIZE):
                ucol = pl.multiple_of((uvec[l] // WIN) * WIN, WIN)
                pltpu.async_copy(
                    ut_hbm.at[:, pl.ds(ucol, WIN)], ubuf.at[l], sem)

        def drain_group(ubuf, sem):
            for l in range(GSIZE):
                pltpu.make_async_copy(
                    ut_hbm.at[:, pl.ds(0, WIN)], ubuf.at[l], sem).wait()

        def compute_group(g, ubuf):
            uvec = uidx[pl.ds(g * GSIZE, LANES)]
            ux = _take16(uvec, item) % WIN
            gitem = g * GSIZE + item
            acc = jnp.zeros((LANES,), jnp.float32)
            for j in range(RANK // GSIZE):
                krow = j * GSIZE + kk
                gu = plsc.load_gather(ubuf, [item, krow, ux])
                gv = plsc.load_gather(vrows, [gitem, krow])
                acc = acc + gu * gv
            acc = acc + _take16(acc, lane ^ 1)
            acc = acc + _take16(acc, lane ^ 2)
            res = _take16(acc, lane * GSIZE)
            plsc.store_compressed(
                outv.at[pl.ds(g * GSIZE, LANES)], res, mask=lane < GSIZE)

        enqueue_group(0, ubufa, sema)

        def step(t, carry):
            g0 = 2 * t
            enqueue_group(g0 + 1, ubufb, semb)
            drain_group(ubufa, sema)
            compute_group(g0, ubufa)

            @pl.when(t < n_groups // 2 - 1)
            def _():
                enqueue_group(g0 + 2, ubufa, sema)
            drain_group(ubufb, semb)
            compute_group(g0 + 1, ubufb)
            return carry

        lax.fori_loop(0, n_groups // 2, step, 0)
        pltpu.sync_copy(outv.at[pl.ds(0, rows_per_w)],
                        out_hbm.at[pl.ds(base, rows_per_w)])

    return body(user2d, Ut, vsel)


def kernel(user, anime, U, V):
    batch = user.shape[0]
    n_workers = 32
    n_chunks = (batch // n_workers) // CHUNK
    user2d = user.astype(jnp.int32).reshape(n_workers, batch // n_workers)
    anime2d = anime.astype(jnp.int32).reshape(n_workers, n_chunks, CHUNK)
    vsel = _v_gather(anime2d, V, n_workers)
    return _u_dot(user2d, U.T, vsel, batch, n_workers)


# speedup vs baseline: 13.8390x; 1.1018x over previous
import functools

import jax
import jax.numpy as jnp
from jax import lax
from jax.experimental import pallas as pl
from jax.experimental.pallas import tpu as pltpu
from jax.experimental.pallas import tpu_sc as plsc

RANK = 32
LANES = 16
GSIZE = 8
WIN = 128
CHUNK = 128


def _take16(vec, idx):
    return vec.at[idx].get(mode="promise_in_bounds")


@functools.partial(jax.jit, static_argnums=(2,))
def _v_gather(anime2d, V, n_workers):
    batch = anime2d.shape[0] * anime2d.shape[1] * anime2d.shape[2]
    rows_per_w = batch // n_workers
    n_chunks = rows_per_w // CHUNK
    mesh = plsc.VectorSubcoreMesh(core_axis_name="c", subcore_axis_name="s")
    info = plsc.get_sparse_core_info()
    nc = info.num_cores

    @functools.partial(
        pl.kernel,
        mesh=mesh,
        compiler_params=pltpu.CompilerParams(
            needs_layout_passes=False, use_tc_tiling_on_sc=False),
        out_type=jax.ShapeDtypeStruct((batch, RANK), jnp.float32),
        scratch_types=[
            pltpu.VMEM((n_chunks, CHUNK), jnp.int32),
            pltpu.VMEM((rows_per_w, RANK), jnp.float32),
            pltpu.SemaphoreType.DMA,
        ],
    )
    def body(anime_hbm, v_hbm, out_hbm, vidx, vrows, sem):
        wid = lax.axis_index("s") * nc + lax.axis_index("c")
        base = pl.multiple_of(wid * rows_per_w, rows_per_w)
        pltpu.sync_copy(anime_hbm.at[wid], vidx)
        copies = []
        for j in range(n_chunks):
            copies.append(pltpu.async_copy(
                v_hbm.at[vidx.at[j]], vrows.at[pl.ds(j * CHUNK, CHUNK)], sem))
        for c in copies:
            c.wait()
        pltpu.sync_copy(vrows, out_hbm.at[pl.ds(base, rows_per_w)])

    return body(anime2d, V)


@functools.partial(jax.jit, static_argnums=(3, 4))
def _u_dot(user2d, Ut, vsel, batch, n_workers):
    rows_per_w = batch // n_workers
    n_groups = rows_per_w // GSIZE
    mesh = plsc.VectorSubcoreMesh(core_axis_name="c", subcore_axis_name="s")
    info = plsc.get_sparse_core_info()
    nc = info.num_cores

    @functools.partial(
        pl.kernel,
        mesh=mesh,
        compiler_params=pltpu.CompilerParams(
            needs_layout_passes=False, use_tc_tiling_on_sc=True),
        out_type=jax.ShapeDtypeStruct((batch,), jnp.float32),
        scratch_types=[
            pltpu.VMEM((rows_per_w + LANES,), jnp.int32),
            pltpu.VMEM((GSIZE, RANK, WIN), jnp.float32),
            pltpu.VMEM((GSIZE, RANK, WIN), jnp.float32),
            pltpu.VMEM((rows_per_w * RANK,), jnp.float32),
            pltpu.VMEM((rows_per_w + LANES,), jnp.float32),
            pltpu.SemaphoreType.DMA,
            pltpu.SemaphoreType.DMA,
        ],
    )
    def body(user_hbm, ut_hbm, vsel_hbm, out_hbm,
             uidx, ubufa, ubufb, vrows, outv, sema, semb):
        wid = lax.axis_index("s") * nc + lax.axis_index("c")
        base = pl.multiple_of(wid * rows_per_w, rows_per_w)

        pltpu.sync_copy(user_hbm.at[wid], uidx.at[pl.ds(0, rows_per_w)])
        pltpu.sync_copy(
            vsel_hbm.at[pl.ds(base * RANK, rows_per_w * RANK)], vrows)

        lane = lax.iota(jnp.int32, LANES)
        kl = LANES // GSIZE
        item = lane // kl
        kk = lane % kl

        def enqueue_group(g, ubuf, sem):
            uvec = uidx[pl.ds(g * GSIZE, LANES)]
            for l in range(GSIZE):
                ucol = pl.multiple_of((uvec[l] // WIN) * WIN, WIN)
                pltpu.async_copy(
                    ut_hbm.at[:, pl.ds(ucol, WIN)], ubuf.at[l], sem)

        def drain_group(ubuf, sem):
            for l in range(GSIZE):
                pltpu.make_async_copy(
                    ut_hbm.at[:, pl.ds(0, WIN)], ubuf.at[l], sem).wait()

        def compute_group(g, ubuf):
            uvec = uidx[pl.ds(g * GSIZE, LANES)]
            ux = _take16(uvec, item) % WIN
            gitem = g * GSIZE + item
            acc = jnp.zeros((LANES,), jnp.float32)
            for j in range(RANK // kl):
                krow = j * kl + kk
                gu = plsc.load_gather(ubuf, [item, krow, ux])
                gv = plsc.load_gather(vrows, [gitem * RANK + krow])
                acc = acc + gu * gv
            fold = 1
            while fold < kl:
                acc = acc + _take16(acc, lane ^ fold)
                fold *= 2
            res = _take16(acc, lane * kl)
            plsc.store_compressed(
                outv.at[pl.ds(g * GSIZE, LANES)], res, mask=lane < GSIZE)

        enqueue_group(0, ubufa, sema)

        def step(t, carry):
            g0 = 2 * t
            enqueue_group(g0 + 1, ubufb, semb)
            drain_group(ubufa, sema)
            compute_group(g0, ubufa)

            @pl.when(t < n_groups // 2 - 1)
            def _():
                enqueue_group(g0 + 2, ubufa, sema)
            drain_group(ubufb, semb)
            compute_group(g0 + 1, ubufb)
            return carry

        lax.fori_loop(0, n_groups // 2, step, 0)
        pltpu.sync_copy(outv.at[pl.ds(0, rows_per_w)],
                        out_hbm.at[pl.ds(base, rows_per_w)])

    return body(user2d, Ut, vsel)


def kernel(user, anime, U, V):
    batch = user.shape[0]
    n_workers = 32
    n_chunks = (batch // n_workers) // CHUNK
    user2d = user.astype(jnp.int32).reshape(n_workers, batch // n_workers)
    anime2d = anime.astype(jnp.int32).reshape(n_workers, n_chunks, CHUNK)
    vsel = _v_gather(anime2d, V, n_workers).reshape(batch * RANK)
    return _u_dot(user2d, U.T, vsel, batch, n_workers)


# speedup vs baseline: 14.2136x; 1.0271x over previous
import functools

import jax
import jax.numpy as jnp
from jax import lax
from jax.experimental import pallas as pl
from jax.experimental.pallas import tpu as pltpu
from jax.experimental.pallas import tpu_sc as plsc

RANK = 32
LANES = 16
GSIZE = 8
WIN = 128
CHUNK = 128


def _take16(vec, idx):
    return vec.at[idx].get(mode="promise_in_bounds")


@functools.partial(jax.jit, static_argnums=(2, 3))
def _u_rows(user2d, Ut, batch, n_workers):
    rows_per_w = batch // n_workers
    n_groups = rows_per_w // GSIZE
    mesh = plsc.VectorSubcoreMesh(core_axis_name="c", subcore_axis_name="s")
    info = plsc.get_sparse_core_info()
    nc = info.num_cores

    @functools.partial(
        pl.kernel,
        mesh=mesh,
        compiler_params=pltpu.CompilerParams(
            needs_layout_passes=False, use_tc_tiling_on_sc=True),
        out_type=jax.ShapeDtypeStruct((batch * RANK,), jnp.float32),
        scratch_types=[
            pltpu.VMEM((rows_per_w + LANES,), jnp.int32),
            pltpu.VMEM((GSIZE, RANK, WIN), jnp.float32),
            pltpu.VMEM((GSIZE, RANK, WIN), jnp.float32),
            pltpu.VMEM((rows_per_w * RANK,), jnp.float32),
            pltpu.SemaphoreType.DMA,
            pltpu.SemaphoreType.DMA,
        ],
    )
    def body(user_hbm, ut_hbm, out_hbm, uidx, ubufa, ubufb, urows,
             sema, semb):
        wid = lax.axis_index("s") * nc + lax.axis_index("c")
        base = pl.multiple_of(wid * rows_per_w, rows_per_w)

        pltpu.sync_copy(user_hbm.at[wid], uidx.at[pl.ds(0, rows_per_w)])

        lane = lax.iota(jnp.int32, LANES)
        kl = LANES // GSIZE
        item = lane // kl
        kk = lane % kl

        def enqueue_group(g, ubuf, sem):
            uvec = uidx[pl.ds(g * GSIZE, LANES)]
            for l in range(GSIZE):
                ucol = pl.multiple_of((uvec[l] // WIN) * WIN, WIN)
                pltpu.async_copy(
                    ut_hbm.at[:, pl.ds(ucol, WIN)], ubuf.at[l], sem)

        def drain_group(ubuf, sem):
            for l in range(GSIZE):
                pltpu.make_async_copy(
                    ut_hbm.at[:, pl.ds(0, WIN)], ubuf.at[l], sem).wait()

        def extract_group(g, ubuf):
            uvec = uidx[pl.ds(g * GSIZE, LANES)]
            ux = _take16(uvec, item) % WIN
            gitem = g * GSIZE + item
            for j in range(RANK // kl):
                krow = j * kl + kk
                gu = plsc.load_gather(ubuf, [item, krow, ux])
                plsc.store_scatter(urows, [gitem * RANK + krow], gu)

        enqueue_group(0, ubufa, sema)

        def step(t, carry):
            g0 = 2 * t
            enqueue_group(g0 + 1, ubufb, semb)
            drain_group(ubufa, sema)
            extract_group(g0, ubufa)

            @pl.when(t < n_groups // 2 - 1)
            def _():
                enqueue_group(g0 + 2, ubufa, sema)
            drain_group(ubufb, semb)
            extract_group(g0 + 1, ubufb)
            return carry

        lax.fori_loop(0, n_groups // 2, step, 0)
        pltpu.sync_copy(urows,
                        out_hbm.at[pl.ds(base * RANK, rows_per_w * RANK)])

    return body(user2d, Ut)


@functools.partial(jax.jit, static_argnums=(3, 4))
def _v_dot(anime2d, V, usel, batch, n_workers):
    rows_per_w = batch // n_workers
    n_chunks = rows_per_w // CHUNK
    n_groups = rows_per_w // LANES
    mesh = plsc.VectorSubcoreMesh(core_axis_name="c", subcore_axis_name="s")
    info = plsc.get_sparse_core_info()
    nc = info.num_cores

    @functools.partial(
        pl.kernel,
        mesh=mesh,
        compiler_params=pltpu.CompilerParams(
            needs_layout_passes=False, use_tc_tiling_on_sc=False),
        out_type=jax.ShapeDtypeStruct((batch,), jnp.float32),
        scratch_types=[
            pltpu.VMEM((n_chunks, CHUNK), jnp.int32),
            pltpu.VMEM((rows_per_w, RANK), jnp.float32),
            pltpu.VMEM((rows_per_w * RANK,), jnp.float32),
            pltpu.VMEM((rows_per_w,), jnp.float32),
            pltpu.SemaphoreType.DMA,
        ],
    )
    def body(anime_hbm, v_hbm, usel_hbm, out_hbm,
             vidx, vrows, urows, outv, sem):
        wid = lax.axis_index("s") * nc + lax.axis_index("c")
        base = pl.multiple_of(wid * rows_per_w, rows_per_w)

        pltpu.sync_copy(anime_hbm.at[wid], vidx)
        copies = [pltpu.async_copy(
            usel_hbm.at[pl.ds(base * RANK, rows_per_w * RANK)], urows, sem)]
        for j in range(n_chunks):
            copies.append(pltpu.async_copy(
                v_hbm.at[vidx.at[j]], vrows.at[pl.ds(j * CHUNK, CHUNK)], sem))
        for c in copies:
            c.wait()

        lane = lax.iota(jnp.int32, LANES)

        def group(g, carry):
            rows = g * LANES + lane
            acc = jnp.zeros((LANES,), jnp.float32)
            for k in range(RANK):
                col = jnp.full((LANES,), k, jnp.int32)
                uu = plsc.load_gather(urows, [rows * RANK + k])
                vv = plsc.load_gather(vrows, [rows, col])
                acc = acc + uu * vv
            outv[pl.ds(g * LANES, LANES)] = acc
            return carry

        lax.fori_loop(0, n_groups, group, 0)
        pltpu.sync_copy(outv, out_hbm.at[pl.ds(base, rows_per_w)])

    return body(anime2d, V, usel)


def kernel(user, anime, U, V):
    batch = user.shape[0]
    n_workers = 32
    n_chunks = (batch // n_workers) // CHUNK
    user2d = user.astype(jnp.int32).reshape(n_workers, batch // n_workers)
    anime2d = anime.astype(jnp.int32).reshape(n_workers, n_chunks, CHUNK)
    usel = _u_rows(user2d, U.T, batch, n_workers)
    return _v_dot(anime2d, V, usel, batch, n_workers)


# speedup vs baseline: 14.7074x; 1.0347x over previous
import functools

import jax
import jax.numpy as jnp
from jax import lax
from jax.experimental import pallas as pl
from jax.experimental.pallas import tpu as pltpu
from jax.experimental.pallas import tpu_sc as plsc

RANK = 32
LANES = 16
GSIZE = 8
WIN = 128
CHUNK = 128


def _take16(vec, idx):
    return vec.at[idx].get(mode="promise_in_bounds")


@functools.partial(jax.jit, static_argnums=(2, 3))
def _u_rows(user2d, Ut, batch, n_workers):
    rows_per_w = batch // n_workers
    n_groups = rows_per_w // GSIZE
    mesh = plsc.VectorSubcoreMesh(core_axis_name="c", subcore_axis_name="s")
    info = plsc.get_sparse_core_info()
    nc = info.num_cores

    @functools.partial(
        pl.kernel,
        mesh=mesh,
        compiler_params=pltpu.CompilerParams(
            needs_layout_passes=False, use_tc_tiling_on_sc=True),
        out_type=jax.ShapeDtypeStruct((batch * RANK,), jnp.float32),
        scratch_types=[
            pltpu.VMEM((rows_per_w + LANES,), jnp.int32),
            pltpu.VMEM((GSIZE, RANK, WIN), jnp.float32),
            pltpu.VMEM((GSIZE, RANK, WIN), jnp.float32),
            pltpu.VMEM((GSIZE, RANK, WIN), jnp.float32),
            pltpu.VMEM((rows_per_w * RANK,), jnp.float32),
            pltpu.SemaphoreType.DMA,
            pltpu.SemaphoreType.DMA,
            pltpu.SemaphoreType.DMA,
        ],
    )
    def body(user_hbm, ut_hbm, out_hbm, uidx, ubufa, ubufb, ubufc, urows,
             sema, semb, semc):
        wid = lax.axis_index("s") * nc + lax.axis_index("c")
        base = pl.multiple_of(wid * rows_per_w, rows_per_w)

        pltpu.sync_copy(user_hbm.at[wid], uidx.at[pl.ds(0, rows_per_w)])

        lane = lax.iota(jnp.int32, LANES)
        kl = LANES // GSIZE
        item = lane // kl
        kk = lane % kl

        def enqueue_group(g, ubuf, sem):
            uvec = uidx[pl.ds(g * GSIZE, LANES)]
            for l in range(GSIZE):
                ucol = pl.multiple_of((uvec[l] // WIN) * WIN, WIN)
                pltpu.async_copy(
                    ut_hbm.at[:, pl.ds(ucol, WIN)], ubuf.at[l], sem)

        def drain_group(ubuf, sem):
            for l in range(GSIZE):
                pltpu.make_async_copy(
                    ut_hbm.at[:, pl.ds(0, WIN)], ubuf.at[l], sem).wait()

        def extract_group(g, ubuf):
            uvec = uidx[pl.ds(g * GSIZE, LANES)]
            ux = _take16(uvec, item) % WIN
            gitem = g * GSIZE + item
            for j in range(RANK // kl):
                krow = j * kl + kk
                gu = plsc.load_gather(ubuf, [item, krow, ux])
                plsc.store_scatter(urows, [gitem * RANK + krow], gu)

        enqueue_group(0, ubufa, sema)
        enqueue_group(1, ubufb, semb)

        def step(t, carry):
            g0 = 3 * t
            enqueue_group(g0 + 2, ubufc, semc)
            drain_group(ubufa, sema)
            extract_group(g0, ubufa)

            @pl.when(t < n_groups // 3 - 1)
            def _():
                enqueue_group(g0 + 3, ubufa, sema)
            drain_group(ubufb, semb)
            extract_group(g0 + 1, ubufb)

            @pl.when(t < n_groups // 3 - 1)
            def _():
                enqueue_group(g0 + 4, ubufb, semb)
            drain_group(ubufc, semc)
            extract_group(g0 + 2, ubufc)
            return carry

        lax.fori_loop(0, n_groups // 3, step, 0)
        for g in range(3 * (n_groups // 3), n_groups):
            enqueue_group(g, ubufa, sema)
            drain_group(ubufa, sema)
            extract_group(g, ubufa)
        pltpu.sync_copy(urows,
                        out_hbm.at[pl.ds(base * RANK, rows_per_w * RANK)])

    return body(user2d, Ut)


@functools.partial(jax.jit, static_argnums=(3, 4))
def _v_dot(anime2d, V, usel, batch, n_workers):
    rows_per_w = batch // n_workers
    n_chunks = rows_per_w // CHUNK
    n_groups = rows_per_w // LANES
    mesh = plsc.VectorSubcoreMesh(core_axis_name="c", subcore_axis_name="s")
    info = plsc.get_sparse_core_info()
    nc = info.num_cores

    @functools.partial(
        pl.kernel,
        mesh=mesh,
        compiler_params=pltpu.CompilerParams(
            needs_layout_passes=False, use_tc_tiling_on_sc=False),
        out_type=jax.ShapeDtypeStruct((batch,), jnp.float32),
        scratch_types=[
            pltpu.VMEM((n_chunks, CHUNK), jnp.int32),
            pltpu.VMEM((rows_per_w, RANK), jnp.float32),
            pltpu.VMEM((rows_per_w * RANK,), jnp.float32),
            pltpu.VMEM((rows_per_w,), jnp.float32),
            pltpu.SemaphoreType.DMA,
        ],
    )
    def body(anime_hbm, v_hbm, usel_hbm, out_hbm,
             vidx, vrows, urows, outv, sem):
        wid = lax.axis_index("s") * nc + lax.axis_index("c")
        base = pl.multiple_of(wid * rows_per_w, rows_per_w)

        pltpu.sync_copy(anime_hbm.at[wid], vidx)
        copies = [pltpu.async_copy(
            usel_hbm.at[pl.ds(base * RANK, rows_per_w * RANK)], urows, sem)]
        for j in range(n_chunks):
            copies.append(pltpu.async_copy(
                v_hbm.at[vidx.at[j]], vrows.at[pl.ds(j * CHUNK, CHUNK)], sem))
        for c in copies:
            c.wait()

        lane = lax.iota(jnp.int32, LANES)

        def group(g, carry):
            rows = g * LANES + lane
            acc = jnp.zeros((LANES,), jnp.float32)
            for k in range(RANK):
                col = jnp.full((LANES,), k, jnp.int32)
                uu = plsc.load_gather(urows, [rows * RANK + k])
                vv = plsc.load_gather(vrows, [rows, col])
                acc = acc + uu * vv
            outv[pl.ds(g * LANES, LANES)] = acc
            return carry

        lax.fori_loop(0, n_groups, group, 0)
        pltpu.sync_copy(outv, out_hbm.at[pl.ds(base, rows_per_w)])

    return body(anime2d, V, usel)


def kernel(user, anime, U, V):
    batch = user.shape[0]
    n_workers = 32
    n_chunks = (batch // n_workers) // CHUNK
    user2d = user.astype(jnp.int32).reshape(n_workers, batch // n_workers)
    anime2d = anime.astype(jnp.int32).reshape(n_workers, n_chunks, CHUNK)
    usel = _u_rows(user2d, U.T, batch, n_workers)
    return _v_dot(anime2d, V, usel, batch, n_workers)
